# Initial kernel scaffold; baseline (speedup 1.0000x reference)
import functools

import jax
import jax.numpy as jnp
from jax import lax
from jax.experimental import pallas as pl
from jax.experimental.pallas import tpu as pltpu
from jax.experimental.pallas import tpu_sc as plsc

N_NODES = 10000
N_EDGES = 320000
D = 128

NC = 2
NS = 16
NW = NC * NS

EB = 128
E_PAD = 327680
NB = E_PAD // (NW * EB)
N_PAD = 10240
RPT = N_PAD // NS


def _mm_body(ns_ref, w_ref, ka_ref, h_ref, sc_ref):
    h = jnp.dot(ns_ref[...], w_ref[...], preferred_element_type=jnp.float32)
    h_ref[...] = h
    a2 = jnp.concatenate([ka_ref[0:D, :], ka_ref[D:2 * D, :]], axis=1)
    sc_ref[...] = jnp.dot(h, a2, preferred_element_type=jnp.float32)


def _transform(node_states, w, ka):
    n = node_states.shape[0]
    blk = 1000
    grid = n // blk
    return pl.pallas_call(
        _mm_body,
        grid=(grid,),
        in_specs=[
            pl.BlockSpec((blk, D), lambda i: (i, 0)),
            pl.BlockSpec((D, D), lambda i: (0, 0)),
            pl.BlockSpec((2 * D, 1), lambda i: (0, 0)),
        ],
        out_specs=[
            pl.BlockSpec((blk, D), lambda i: (i, 0)),
            pl.BlockSpec((blk, 2), lambda i: (i, 0)),
        ],
        out_shape=[
            jax.ShapeDtypeStruct((n, D), jnp.float32),
            jax.ShapeDtypeStruct((n, 2), jnp.float32),
        ],
    )(node_states, w, ka)


def _sc_body(tgt_hbm, src_hbm, st_hbm, ss_hbm, h_hbm,
             acc_hbm, sum_hbm,
             tgt_v, src_v, st_v, ss_v, e_v, rows_v, zsum_v,
             acc_sh, sum_sh, sem):
    c = lax.axis_index("c")
    s = lax.axis_index("s")
    wid = c * NS + s

    pltpu.sync_copy(st_hbm, st_v)
    pltpu.sync_copy(ss_hbm, ss_v)
    pltpu.sync_copy(tgt_hbm.at[pl.ds(wid * NB, NB)], tgt_v)
    pltpu.sync_copy(src_hbm.at[pl.ds(wid * NB, NB)], src_v)

    zero16 = jnp.zeros((16,), jnp.float32)

    def zrow(j, carry):
        for k in range(D // 16):
            rows_v[j, pl.ds(k * 16, 16)] = zero16
        return carry

    lax.fori_loop(0, EB, zrow, 0)

    def zs(j, carry):
        zsum_v[pl.ds(j * 16, 16)] = zero16
        return carry

    lax.fori_loop(0, RPT // 16, zs, 0)

    for r in range(RPT // EB):
        pltpu.sync_copy(rows_v, acc_sh.at[pl.ds(s * RPT + r * EB, EB)])
    pltpu.sync_copy(zsum_v, sum_sh.at[pl.ds(s * RPT, RPT)])
    plsc.subcore_barrier()

    def batch_body(b, carry):
        pltpu.async_copy(h_hbm.at[src_v.at[b]], rows_v, sem).wait()

        for k in range(EB // 16):
            t16 = tgt_v[b, pl.ds(k * 16, 16)]
            s16 = src_v[b, pl.ds(k * 16, 16)]
            raw = plsc.load_gather(st_v, [t16]) + plsc.load_gather(ss_v, [s16])
            lr = jnp.where(raw >= 0.0, raw, raw * 0.2)
            e_v[b, pl.ds(k * 16, 16)] = jnp.exp(jnp.clip(lr, -2.0, 2.0))

        def scale_row(j, carry2):
            av = jnp.full((16,), e_v[b, j], jnp.float32)
            for k in range(D // 16):
                rows_v[j, pl.ds(k * 16, 16)] = rows_v[j, pl.ds(k * 16, 16)] * av
            return carry2

        lax.fori_loop(0, EB, scale_row, 0)

        pltpu.sync_copy(rows_v, acc_sh.at[tgt_v.at[b]], add=True)
        pltpu.sync_copy(e_v.at[b], sum_sh.at[tgt_v.at[b]], add=True)
        return carry

    lax.fori_loop(0, NB, batch_body, 0)
    plsc.subcore_barrier()

    for r in range(RPT // EB):
        pltpu.sync_copy(acc_sh.at[pl.ds(s * RPT + r * EB, EB)],
                        acc_hbm.at[c, pl.ds(s * RPT + r * EB, EB)])
    pltpu.sync_copy(sum_sh.at[pl.ds(s * RPT, RPT)],
                    sum_hbm.at[c, pl.ds(s * RPT, RPT)])


def _sc_edge_pass(tgt2d, src2d, st_pad, ss_pad, h):
    mesh = plsc.VectorSubcoreMesh(core_axis_name="c", subcore_axis_name="s")
    return pl.kernel(
        _sc_body,
        out_type=[
            jax.ShapeDtypeStruct((NC, N_PAD, D), jnp.float32),
            jax.ShapeDtypeStruct((NC, N_PAD), jnp.float32),
        ],
        mesh=mesh,
        scratch_types=[
            pltpu.VMEM((NB, EB), jnp.int32),
            pltpu.VMEM((NB, EB), jnp.int32),
            pltpu.VMEM((N_PAD,), jnp.float32),
            pltpu.VMEM((N_PAD,), jnp.float32),
            pltpu.VMEM((NB, EB), jnp.float32),
            pltpu.VMEM((EB, D), jnp.float32),
            pltpu.VMEM((RPT,), jnp.float32),
            pltpu.VMEM_SHARED((N_PAD, D), jnp.float32),
            pltpu.VMEM_SHARED((N_PAD,), jnp.float32),
            pltpu.SemaphoreType.DMA,
        ],
    )(tgt2d, src2d, st_pad, ss_pad, h)


def _norm_body(a0_ref, a1_ref, s0_ref, s1_ref, o_ref):
    tot = s0_ref[...] + s1_ref[...] + 1e-9
    o_ref[...] = (a0_ref[...] + a1_ref[...]) / tot


def _normalize(acc, sums):
    blk = 1024
    grid = N_PAD // blk
    return pl.pallas_call(
        _norm_body,
        grid=(grid,),
        in_specs=[
            pl.BlockSpec((blk, D), lambda i: (i, 0)),
            pl.BlockSpec((blk, D), lambda i: (i, 0)),
            pl.BlockSpec((blk, 1), lambda i: (i, 0)),
            pl.BlockSpec((blk, 1), lambda i: (i, 0)),
        ],
        out_specs=pl.BlockSpec((blk, D), lambda i: (i, 0)),
        out_shape=jax.ShapeDtypeStruct((N_PAD, D), jnp.float32),
    )(acc[0], acc[1], sums[0][:, None], sums[1][:, None])


@jax.jit
def kernel(node_states, edges, kernel, kernel_attention):
    h, scores = _transform(node_states, kernel, kernel_attention)

    tgt = edges[:, 0].astype(jnp.int32)
    src = edges[:, 1].astype(jnp.int32)
    n_fill = E_PAD - N_EDGES
    fill_t = N_NODES + (jnp.arange(n_fill, dtype=jnp.int32) % (N_PAD - N_NODES))
    tgt2d = jnp.concatenate([tgt, fill_t]).reshape(NW * NB, EB)
    src2d = jnp.concatenate([src, jnp.zeros((n_fill,), jnp.int32)]).reshape(
        NW * NB, EB)

    st_pad = jnp.pad(scores[:, 0], (0, N_PAD - N_NODES))
    ss_pad = jnp.pad(scores[:, 1], (0, N_PAD - N_NODES))

    acc, sums = _sc_edge_pass(tgt2d, src2d, st_pad, ss_pad, h)
    out = _normalize(acc, sums)
    return out[:N_NODES]


# speedup vs baseline: 7.5062x; 7.5062x over previous
import functools

import jax
import jax.numpy as jnp
from jax import lax
from jax.experimental import pallas as pl
from jax.experimental.pallas import tpu as pltpu
from jax.experimental.pallas import tpu_sc as plsc

N_NODES = 10000
N_EDGES = 320000
D = 128

NC = 2
NS = 16
NW = NC * NS

EB = 128
E_PAD = 327680
NB = E_PAD // (NW * EB)
N_PAD = 10240
RPT = N_PAD // NS
G = 8


def _mm_body(ns_ref, w_ref, ka_ref, h_ref, sc_ref):
    h = jnp.dot(ns_ref[...], w_ref[...], preferred_element_type=jnp.float32)
    h_ref[...] = h
    a2 = jnp.concatenate([ka_ref[0:D, :], ka_ref[D:2 * D, :]], axis=1)
    sc_ref[...] = jnp.dot(h, a2, preferred_element_type=jnp.float32)


def _transform(node_states, w, ka):
    n = node_states.shape[0]
    blk = 1000
    grid = n // blk
    return pl.pallas_call(
        _mm_body,
        grid=(grid,),
        in_specs=[
            pl.BlockSpec((blk, D), lambda i: (i, 0)),
            pl.BlockSpec((D, D), lambda i: (0, 0)),
            pl.BlockSpec((2 * D, 1), lambda i: (0, 0)),
        ],
        out_specs=[
            pl.BlockSpec((blk, D), lambda i: (i, 0)),
            pl.BlockSpec((blk, 2), lambda i: (i, 0)),
        ],
        out_shape=[
            jax.ShapeDtypeStruct((n, D), jnp.float32),
            jax.ShapeDtypeStruct((n, 2), jnp.float32),
        ],
    )(node_states, w, ka)


def _sc_body(tgt_hbm, src_hbm, st_hbm, ss_hbm, h_hbm,
             acc_hbm, sum_hbm,
             tgt_v, src_v, st_v, ss_v, e_v, rows_v, zsum_v,
             acc_sh, sum_sh, sem):
    c = lax.axis_index("c")
    s = lax.axis_index("s")
    wid = c * NS + s

    pltpu.sync_copy(st_hbm, st_v)
    pltpu.sync_copy(ss_hbm, ss_v)

    zero16 = jnp.zeros((16,), jnp.float32)

    def zrow(j, carry):
        for k in range(D // 16):
            rows_v[j, pl.ds(k * 16, 16)] = zero16
        return carry

    lax.fori_loop(0, EB, zrow, 0)

    def zs(j, carry):
        zsum_v[pl.ds(j * 16, 16)] = zero16
        return carry

    lax.fori_loop(0, RPT // 16, zs, 0)

    for r in range(RPT // EB):
        pltpu.sync_copy(rows_v, acc_sh.at[pl.ds(s * RPT + r * EB, EB)])
    pltpu.sync_copy(zsum_v, sum_sh.at[pl.ds(s * RPT, RPT)])
    plsc.subcore_barrier()

    def group_body(go, carry):
        base_row = wid * NB + go * G
        pltpu.sync_copy(tgt_hbm.at[pl.ds(base_row, G)], tgt_v)
        pltpu.sync_copy(src_hbm.at[pl.ds(base_row, G)], src_v)

        def batch_body(g, carry1):
            pltpu.async_copy(h_hbm.at[src_v.at[g]], rows_v, sem).wait()

            def chunk(k, carry2):
                base = k * 16
                t16 = tgt_v[g, pl.ds(base, 16)]
                s16 = src_v[g, pl.ds(base, 16)]
                raw = (plsc.load_gather(st_v, [t16]) +
                       plsc.load_gather(ss_v, [s16]))
                lr = jnp.where(raw >= 0.0, raw, raw * 0.2)
                e16 = jnp.exp(jnp.clip(lr, -2.0, 2.0))
                e_v[g, pl.ds(base, 16)] = e16
                for lane in range(16):
                    av = jnp.full((16,), e16[lane], jnp.float32)
                    j = base + lane
                    for kk in range(D // 16):
                        rows_v[j, pl.ds(kk * 16, 16)] = (
                            rows_v[j, pl.ds(kk * 16, 16)] * av)
                return carry2

            lax.fori_loop(0, EB // 16, chunk, 0)

            pltpu.sync_copy(rows_v, acc_sh.at[tgt_v.at[g]], add=True)
            pltpu.sync_copy(e_v.at[g], sum_sh.at[tgt_v.at[g]], add=True)
            return carry1

        lax.fori_loop(0, G, batch_body, 0)
        return carry

    lax.fori_loop(0, NB // G, group_body, 0)
    plsc.subcore_barrier()

    for r in range(RPT // EB):
        pltpu.sync_copy(acc_sh.at[pl.ds(s * RPT + r * EB, EB)],
                        acc_hbm.at[c, pl.ds(s * RPT + r * EB, EB)])
    pltpu.sync_copy(sum_sh.at[pl.ds(s * RPT, RPT)],
                    sum_hbm.at[c, pl.ds(s * RPT, RPT)])


def _sc_edge_pass(tgt2d, src2d, st_pad, ss_pad, h):
    mesh = plsc.VectorSubcoreMesh(core_axis_name="c", subcore_axis_name="s")
    return pl.kernel(
        _sc_body,
        out_type=[
            jax.ShapeDtypeStruct((NC, N_PAD, D), jnp.float32),
            jax.ShapeDtypeStruct((NC, N_PAD), jnp.float32),
        ],
        mesh=mesh,
        compiler_params=pltpu.CompilerParams(needs_layout_passes=False),
        scratch_types=[
            pltpu.VMEM((G, EB), jnp.int32),
            pltpu.VMEM((G, EB), jnp.int32),
            pltpu.VMEM((N_PAD,), jnp.float32),
            pltpu.VMEM((N_PAD,), jnp.float32),
            pltpu.VMEM((G, EB), jnp.float32),
            pltpu.VMEM((EB, D), jnp.float32),
            pltpu.VMEM((RPT,), jnp.float32),
            pltpu.VMEM_SHARED((N_PAD, D), jnp.float32),
            pltpu.VMEM_SHARED((N_PAD,), jnp.float32),
            pltpu.SemaphoreType.DMA,
        ],
    )(tgt2d, src2d, st_pad, ss_pad, h)


def _norm_body(a0_ref, a1_ref, s0_ref, s1_ref, o_ref):
    tot = s0_ref[...] + s1_ref[...] + 1e-9
    o_ref[...] = (a0_ref[...] + a1_ref[...]) / tot


def _normalize(acc, sums):
    blk = 1024
    grid = N_PAD // blk
    return pl.pallas_call(
        _norm_body,
        grid=(grid,),
        in_specs=[
            pl.BlockSpec((blk, D), lambda i: (i, 0)),
            pl.BlockSpec((blk, D), lambda i: (i, 0)),
            pl.BlockSpec((blk, 1), lambda i: (i, 0)),
            pl.BlockSpec((blk, 1), lambda i: (i, 0)),
        ],
        out_specs=pl.BlockSpec((blk, D), lambda i: (i, 0)),
        out_shape=jax.ShapeDtypeStruct((N_PAD, D), jnp.float32),
    )(acc[0], acc[1], sums[0][:, None], sums[1][:, None])


@jax.jit
def kernel(node_states, edges, kernel, kernel_attention):
    h, scores = _transform(node_states, kernel, kernel_attention)

    tgt = edges[:, 0].astype(jnp.int32)
    src = edges[:, 1].astype(jnp.int32)
    n_fill = E_PAD - N_EDGES
    fill_t = N_NODES + (jnp.arange(n_fill, dtype=jnp.int32) % (N_PAD - N_NODES))
    tgt2d = jnp.concatenate([tgt, fill_t]).reshape(NW * NB, EB)
    src2d = jnp.concatenate([src, jnp.zeros((n_fill,), jnp.int32)]).reshape(
        NW * NB, EB)

    st_pad = jnp.pad(scores[:, 0], (0, N_PAD - N_NODES))
    ss_pad = jnp.pad(scores[:, 1], (0, N_PAD - N_NODES))

    acc, sums = _sc_edge_pass(tgt2d, src2d, st_pad, ss_pad, h)
    out = _normalize(acc, sums)
    return out[:N_NODES]


# speedup vs baseline: 8.9570x; 1.1933x over previous
import functools

import jax
import jax.numpy as jnp
from jax import lax
from jax.experimental import pallas as pl
from jax.experimental.pallas import tpu as pltpu
from jax.experimental.pallas import tpu_sc as plsc

N_NODES = 10000
N_EDGES = 320000
D = 128

NC = 2
NS = 16
NW = NC * NS

EB = 128
E_PAD = 327680
NB = E_PAD // (NW * EB)
N_PAD = 10240
RPT = N_PAD // NS
G = 8


def _mm_body(ns_ref, w_ref, ka_ref, h_ref, sc_ref):
    h = jnp.dot(ns_ref[...], w_ref[...], preferred_element_type=jnp.float32)
    h_ref[...] = h
    a2 = jnp.concatenate([ka_ref[0:D, :], ka_ref[D:2 * D, :]], axis=1)
    sc_ref[...] = jnp.dot(h, a2, preferred_element_type=jnp.float32)


def _transform(node_states, w, ka):
    n = node_states.shape[0]
    blk = 1000
    grid = n // blk
    return pl.pallas_call(
        _mm_body,
        grid=(grid,),
        in_specs=[
            pl.BlockSpec((blk, D), lambda i: (i, 0)),
            pl.BlockSpec((D, D), lambda i: (0, 0)),
            pl.BlockSpec((2 * D, 1), lambda i: (0, 0)),
        ],
        out_specs=[
            pl.BlockSpec((blk, D), lambda i: (i, 0)),
            pl.BlockSpec((blk, 2), lambda i: (i, 0)),
        ],
        out_shape=[
            jax.ShapeDtypeStruct((n, D), jnp.float32),
            jax.ShapeDtypeStruct((n, 2), jnp.float32),
        ],
    )(node_states, w, ka)


def _sc_body(tgt_hbm, src_hbm, st_hbm, ss_hbm, h_hbm,
             acc_hbm, sum_hbm,
             tgt_v, src_v, stg_v, ssg_v, e_v, rows_v, rows_w, zsum_v,
             acc_sh, sum_sh, sem0, sem1):
    c = lax.axis_index("c")
    s = lax.axis_index("s")
    wid = c * NS + s

    zero16 = jnp.zeros((16,), jnp.float32)

    def zrow(j, carry):
        for k in range(D // 16):
            rows_v[j, pl.ds(k * 16, 16)] = zero16
        return carry

    lax.fori_loop(0, EB, zrow, 0)

    def zs(j, carry):
        zsum_v[pl.ds(j * 16, 16)] = zero16
        return carry

    lax.fori_loop(0, RPT // 16, zs, 0)

    for r in range(RPT // EB):
        pltpu.sync_copy(rows_v, acc_sh.at[pl.ds(s * RPT + r * EB, EB)])
    pltpu.sync_copy(zsum_v, sum_sh.at[pl.ds(s * RPT, RPT)])
    plsc.subcore_barrier()

    def compute_e(g, par):
        def chunk(k, carry2):
            base = k * 16
            raw = stg_v[par, pl.ds(base, 16)] + ssg_v[par, pl.ds(base, 16)]
            lr = jnp.where(raw >= 0.0, raw, raw * 0.2)
            e_v[g, pl.ds(base, 16)] = jnp.exp(jnp.clip(lr, -2.0, 2.0))
            return carry2

        lax.fori_loop(0, EB // 16, chunk, 0)

    def scale_and_scatter(g, buf):
        def chunk(k, carry2):
            base = k * 16
            e16 = e_v[g, pl.ds(base, 16)]
            for lane in range(16):
                av = jnp.full((16,), e16[lane], jnp.float32)
                j = base + lane
                for kk in range(D // 16):
                    buf[j, pl.ds(kk * 16, 16)] = buf[j, pl.ds(kk * 16, 16)] * av
            return carry2

        lax.fori_loop(0, EB // 16, chunk, 0)
        pltpu.sync_copy(buf, acc_sh.at[tgt_v.at[g]], add=True)
        pltpu.sync_copy(e_v.at[g], sum_sh.at[tgt_v.at[g]], add=True)

    def issue_gathers(g, par, buf, sem_):
        pltpu.async_copy(h_hbm.at[src_v.at[g]], buf, sem_)
        pltpu.async_copy(st_hbm.at[tgt_v.at[g]], stg_v.at[par], sem_)
        pltpu.async_copy(ss_hbm.at[src_v.at[g]], ssg_v.at[par], sem_)

    def wait_gathers(g, par, buf, sem_):
        pltpu.make_async_copy(h_hbm.at[src_v.at[g]], buf, sem_).wait()
        pltpu.make_async_copy(st_hbm.at[tgt_v.at[g]], stg_v.at[par], sem_).wait()
        pltpu.make_async_copy(ss_hbm.at[src_v.at[g]], ssg_v.at[par], sem_).wait()

    def group_body(go, carry):
        base_row = wid * NB + go * G
        pltpu.sync_copy(tgt_hbm.at[pl.ds(base_row, G)], tgt_v)
        pltpu.sync_copy(src_hbm.at[pl.ds(base_row, G)], src_v)

        issue_gathers(0, 0, rows_v, sem0)

        def pair_body(p, carry1):
            b0 = 2 * p
            b1 = b0 + 1
            issue_gathers(b1, 1, rows_w, sem1)
            wait_gathers(b0, 0, rows_v, sem0)
            compute_e(b0, 0)
            scale_and_scatter(b0, rows_v)

            @pl.when(p < G // 2 - 1)
            def _():
                issue_gathers(b1 + 1, 0, rows_v, sem0)

            wait_gathers(b1, 1, rows_w, sem1)
            compute_e(b1, 1)
            scale_and_scatter(b1, rows_w)
            return carry1

        lax.fori_loop(0, G // 2, pair_body, 0)
        return carry

    lax.fori_loop(0, NB // G, group_body, 0)
    plsc.subcore_barrier()

    for r in range(RPT // EB):
        pltpu.sync_copy(acc_sh.at[pl.ds(s * RPT + r * EB, EB)],
                        acc_hbm.at[c, pl.ds(s * RPT + r * EB, EB)])
    pltpu.sync_copy(sum_sh.at[pl.ds(s * RPT, RPT)],
                    sum_hbm.at[c, pl.ds(s * RPT, RPT)])


def _sc_edge_pass(tgt2d, src2d, st_pad, ss_pad, h):
    mesh = plsc.VectorSubcoreMesh(core_axis_name="c", subcore_axis_name="s")
    return pl.kernel(
        _sc_body,
        out_type=[
            jax.ShapeDtypeStruct((NC, N_PAD, D), jnp.float32),
            jax.ShapeDtypeStruct((NC, N_PAD), jnp.float32),
        ],
        mesh=mesh,
        compiler_params=pltpu.CompilerParams(needs_layout_passes=False),
        scratch_types=[
            pltpu.VMEM((G, EB), jnp.int32),
            pltpu.VMEM((G, EB), jnp.int32),
            pltpu.VMEM((2, EB), jnp.float32),
            pltpu.VMEM((2, EB), jnp.float32),
            pltpu.VMEM((G, EB), jnp.float32),
            pltpu.VMEM((EB, D), jnp.float32),
            pltpu.VMEM((EB, D), jnp.float32),
            pltpu.VMEM((RPT,), jnp.float32),
            pltpu.VMEM_SHARED((N_PAD, D), jnp.float32),
            pltpu.VMEM_SHARED((N_PAD,), jnp.float32),
            pltpu.SemaphoreType.DMA,
            pltpu.SemaphoreType.DMA,
        ],
    )(tgt2d, src2d, st_pad, ss_pad, h)


def _norm_body(a0_ref, a1_ref, s0_ref, s1_ref, o_ref):
    tot = s0_ref[...] + s1_ref[...] + 1e-9
    o_ref[...] = (a0_ref[...] + a1_ref[...]) / tot


def _normalize(acc, sums):
    blk = 1024
    grid = N_PAD // blk
    return pl.pallas_call(
        _norm_body,
        grid=(grid,),
        in_specs=[
            pl.BlockSpec((blk, D), lambda i: (i, 0)),
            pl.BlockSpec((blk, D), lambda i: (i, 0)),
            pl.BlockSpec((blk, 1), lambda i: (i, 0)),
            pl.BlockSpec((blk, 1), lambda i: (i, 0)),
        ],
        out_specs=pl.BlockSpec((blk, D), lambda i: (i, 0)),
        out_shape=jax.ShapeDtypeStruct((N_PAD, D), jnp.float32),
    )(acc[0], acc[1], sums[0][:, None], sums[1][:, None])


@jax.jit
def kernel(node_states, edges, kernel, kernel_attention):
    h, scores = _transform(node_states, kernel, kernel_attention)

    tgt = edges[:, 0].astype(jnp.int32)
    src = edges[:, 1].astype(jnp.int32)
    n_fill = E_PAD - N_EDGES
    fill_t = N_NODES + (jnp.arange(n_fill, dtype=jnp.int32) % (N_PAD - N_NODES))
    tgt2d = jnp.concatenate([tgt, fill_t]).reshape(NW * NB, EB)
    src2d = jnp.concatenate([src, jnp.zeros((n_fill,), jnp.int32)]).reshape(
        NW * NB, EB)

    st_pad = jnp.pad(scores[:, 0], (0, N_PAD - N_NODES))
    ss_pad = jnp.pad(scores[:, 1], (0, N_PAD - N_NODES))

    acc, sums = _sc_edge_pass(tgt2d, src2d, st_pad, ss_pad, h)
    out = _normalize(acc, sums)
    return out[:N_NODES]


# speedup vs baseline: 10.4885x; 1.1710x over previous
import functools

import numpy as np
import jax
import jax.numpy as jnp
from jax import lax
from jax.experimental import pallas as pl
from jax.experimental.pallas import tpu as pltpu
from jax.experimental.pallas import tpu_sc as plsc

N_NODES = 10000
N_EDGES = 320000
D = 128

NC = 2
NS = 16
NW = NC * NS

EB = 128
E_PAD = 327680
NB = E_PAD // (NW * EB)
N_PAD = 10240
RPT = N_PAD // NS
G = 8


def _mm_body(ns_ref, w_ref, ka_ref, h_ref, sc_ref):
    h = jnp.dot(ns_ref[...], w_ref[...], preferred_element_type=jnp.float32)
    h_ref[...] = h.astype(jnp.bfloat16)
    a2 = jnp.concatenate([ka_ref[0:D, :], ka_ref[D:2 * D, :]], axis=1)
    sc_ref[...] = jnp.dot(h, a2, preferred_element_type=jnp.float32)


def _transform(node_states, w, ka):
    n = node_states.shape[0]
    blk = 1000
    grid = n // blk
    return pl.pallas_call(
        _mm_body,
        grid=(grid,),
        in_specs=[
            pl.BlockSpec((blk, D), lambda i: (i, 0)),
            pl.BlockSpec((D, D), lambda i: (0, 0)),
            pl.BlockSpec((2 * D, 1), lambda i: (0, 0)),
        ],
        out_specs=[
            pl.BlockSpec((blk, D), lambda i: (i, 0)),
            pl.BlockSpec((blk, 2), lambda i: (i, 0)),
        ],
        out_shape=[
            jax.ShapeDtypeStruct((n, D), jnp.bfloat16),
            jax.ShapeDtypeStruct((n, 2), jnp.float32),
        ],
    )(node_states, w, ka)


def _sc_body(tgt_hbm, src_hbm, st_hbm, ss_hbm, h_hbm,
             acc_hbm, sum_hbm,
             tgt_v, src_v, stg_v, ssg_v, e_v, rows_v, rows_w, sbuf, zsum_v,
             acc_sh, sum_sh, sem0, sem1):
    c = lax.axis_index("c")
    s = lax.axis_index("s")
    wid = c * NS + s

    zero16 = jnp.zeros((16,), jnp.float32)

    def zrow(j, carry):
        for k in range(D // 16):
            sbuf[j, pl.ds(k * 16, 16)] = zero16
        return carry

    lax.fori_loop(0, EB, zrow, 0)

    def zs(j, carry):
        zsum_v[pl.ds(j * 16, 16)] = zero16
        return carry

    lax.fori_loop(0, RPT // 16, zs, 0)

    for r in range(RPT // EB):
        pltpu.sync_copy(sbuf, acc_sh.at[pl.ds(s * RPT + r * EB, EB)])
    pltpu.sync_copy(zsum_v, sum_sh.at[pl.ds(s * RPT, RPT)])
    plsc.subcore_barrier()

    def compute_e(g, par):
        def chunk(k, carry2):
            base = k * 16
            raw = stg_v[par, pl.ds(base, 16)] + ssg_v[par, pl.ds(base, 16)]
            lr = jnp.where(raw >= 0.0, raw, raw * 0.2)
            e_v[g, pl.ds(base, 16)] = jnp.exp(jnp.clip(lr, -2.0, 2.0))
            return carry2

        lax.fori_loop(0, EB // 16, chunk, 0)

    def scale_and_scatter(g, buf):
        hi_mask = jnp.full((16,), 0xFFFF0000, jnp.uint32)

        def chunk(k, carry2):
            base = k * 16
            e16 = e_v[g, pl.ds(base, 16)]
            for lane in range(16):
                av = jnp.full((16,), e16[lane], jnp.float32)
                j = base + lane
                for kk in range(D // 32):
                    w = plsc.bitcast(buf[j, pl.ds(kk * 16, 16)], jnp.uint32)
                    lo = plsc.bitcast(w << 16, jnp.float32)
                    hi = plsc.bitcast(w & hi_mask, jnp.float32)
                    sbuf[j, pl.ds(kk * 32, 16)] = lo * av
                    sbuf[j, pl.ds(kk * 32 + 16, 16)] = hi * av
            return carry2

        lax.fori_loop(0, EB // 16, chunk, 0)
        pltpu.sync_copy(sbuf, acc_sh.at[tgt_v.at[g]], add=True)
        pltpu.sync_copy(e_v.at[g], sum_sh.at[tgt_v.at[g]], add=True)

    def issue_gathers(g, par, buf, sem_):
        pltpu.async_copy(h_hbm.at[src_v.at[g]], buf, sem_)
        pltpu.async_copy(st_hbm.at[tgt_v.at[g]], stg_v.at[par], sem_)
        pltpu.async_copy(ss_hbm.at[src_v.at[g]], ssg_v.at[par], sem_)

    def wait_gathers(g, par, buf, sem_):
        pltpu.make_async_copy(h_hbm.at[src_v.at[g]], buf, sem_).wait()
        pltpu.make_async_copy(st_hbm.at[tgt_v.at[g]], stg_v.at[par], sem_).wait()
        pltpu.make_async_copy(ss_hbm.at[src_v.at[g]], ssg_v.at[par], sem_).wait()

    def group_body(go, carry):
        base_row = wid * NB + go * G
        pltpu.sync_copy(tgt_hbm.at[pl.ds(base_row, G)], tgt_v)
        pltpu.sync_copy(src_hbm.at[pl.ds(base_row, G)], src_v)

        issue_gathers(0, 0, rows_v, sem0)

        def pair_body(p, carry1):
            b0 = 2 * p
            b1 = b0 + 1
            issue_gathers(b1, 1, rows_w, sem1)
            wait_gathers(b0, 0, rows_v, sem0)
            compute_e(b0, 0)
            scale_and_scatter(b0, rows_v)

            @pl.when(p < G // 2 - 1)
            def _():
                issue_gathers(b1 + 1, 0, rows_v, sem0)

            wait_gathers(b1, 1, rows_w, sem1)
            compute_e(b1, 1)
            scale_and_scatter(b1, rows_w)
            return carry1

        lax.fori_loop(0, G // 2, pair_body, 0)
        return carry

    lax.fori_loop(0, NB // G, group_body, 0)
    plsc.subcore_barrier()

    for r in range(RPT // EB):
        pltpu.sync_copy(acc_sh.at[pl.ds(s * RPT + r * EB, EB)],
                        acc_hbm.at[c, pl.ds(s * RPT + r * EB, EB)])
    pltpu.sync_copy(sum_sh.at[pl.ds(s * RPT, RPT)],
                    sum_hbm.at[c, pl.ds(s * RPT, RPT)])


def _sc_edge_pass(tgt2d, src2d, st_pad, ss_pad, h):
    mesh = plsc.VectorSubcoreMesh(core_axis_name="c", subcore_axis_name="s")
    return pl.kernel(
        _sc_body,
        out_type=[
            jax.ShapeDtypeStruct((NC, N_PAD, D), jnp.float32),
            jax.ShapeDtypeStruct((NC, N_PAD), jnp.float32),
        ],
        mesh=mesh,
        compiler_params=pltpu.CompilerParams(needs_layout_passes=False,
                                             use_tc_tiling_on_sc=False),
        scratch_types=[
            pltpu.VMEM((G, EB), jnp.int32),
            pltpu.VMEM((G, EB), jnp.int32),
            pltpu.VMEM((2, EB), jnp.float32),
            pltpu.VMEM((2, EB), jnp.float32),
            pltpu.VMEM((G, EB), jnp.float32),
            pltpu.VMEM((EB, D // 2), jnp.int32),
            pltpu.VMEM((EB, D // 2), jnp.int32),
            pltpu.VMEM((EB, D), jnp.float32),
            pltpu.VMEM((RPT,), jnp.float32),
            pltpu.VMEM_SHARED((N_PAD, D), jnp.float32),
            pltpu.VMEM_SHARED((N_PAD,), jnp.float32),
            pltpu.SemaphoreType.DMA,
            pltpu.SemaphoreType.DMA,
        ],
    )(tgt2d, src2d, st_pad, ss_pad, h)


def _norm_body(a0_ref, a1_ref, s0_ref, s1_ref, p_ref, o_ref):
    tot = s0_ref[...] + s1_ref[...] + 1e-9
    acc = jnp.dot(a0_ref[...] + a1_ref[...], p_ref[...],
                  preferred_element_type=jnp.float32)
    o_ref[...] = acc / tot


def _normalize(acc, sums, p):
    blk = 1024
    grid = N_PAD // blk
    return pl.pallas_call(
        _norm_body,
        grid=(grid,),
        in_specs=[
            pl.BlockSpec((blk, D), lambda i: (i, 0)),
            pl.BlockSpec((blk, D), lambda i: (i, 0)),
            pl.BlockSpec((blk, 1), lambda i: (i, 0)),
            pl.BlockSpec((blk, 1), lambda i: (i, 0)),
            pl.BlockSpec((D, D), lambda i: (0, 0)),
        ],
        out_specs=pl.BlockSpec((blk, D), lambda i: (i, 0)),
        out_shape=jax.ShapeDtypeStruct((N_PAD, D), jnp.float32),
    )(acc[0], acc[1], sums[0][:, None], sums[1][:, None], p)


_SC_PERM = np.concatenate(
    [np.concatenate([np.arange(32 * kk, 32 * (kk + 1), 2),
                     np.arange(32 * kk + 1, 32 * (kk + 1), 2)])
     for kk in range(D // 32)])
_P_UNPERM = np.zeros((D, D), np.float32)
_P_UNPERM[np.arange(D), _SC_PERM] = 1.0


@jax.jit
def kernel(node_states, edges, kernel, kernel_attention):
    h, scores = _transform(node_states, kernel, kernel_attention)

    h32 = lax.bitcast_convert_type(
        h.reshape(h.shape[0], D // 2, 2), jnp.int32)

    tgt = edges[:, 0].astype(jnp.int32)
    src = edges[:, 1].astype(jnp.int32)
    n_fill = E_PAD - N_EDGES
    fill_t = N_NODES + (jnp.arange(n_fill, dtype=jnp.int32) % (N_PAD - N_NODES))
    tgt2d = jnp.concatenate([tgt, fill_t]).reshape(NW * NB, EB)
    src2d = jnp.concatenate([src, jnp.zeros((n_fill,), jnp.int32)]).reshape(
        NW * NB, EB)

    st_pad = jnp.pad(scores[:, 0], (0, N_PAD - N_NODES))
    ss_pad = jnp.pad(scores[:, 1], (0, N_PAD - N_NODES))

    acc, sums = _sc_edge_pass(tgt2d, src2d, st_pad, ss_pad, h32)
    out = _normalize(acc, sums, jnp.asarray(_P_UNPERM))
    return out[:N_NODES]


# speedup vs baseline: 10.9314x; 1.0422x over previous
import functools

import numpy as np
import jax
import jax.numpy as jnp
from jax import lax
from jax.experimental import pallas as pl
from jax.experimental.pallas import tpu as pltpu
from jax.experimental.pallas import tpu_sc as plsc

N_NODES = 10000
N_EDGES = 320000
D = 128

NC = 2
NS = 16
NW = NC * NS

EB = 128
E_PAD = 327680
NB = E_PAD // (NW * EB)
N_PAD = 10240
RPT = N_PAD // NS
G = 8


def _mm_body(ns_ref, w_ref, ka_ref, h_ref, sc_ref):
    h = jnp.dot(ns_ref[...], w_ref[...], preferred_element_type=jnp.float32)
    h_ref[...] = h.astype(jnp.bfloat16)
    a2 = jnp.concatenate([ka_ref[0:D, :], ka_ref[D:2 * D, :]], axis=1)
    sc_ref[...] = jnp.dot(h, a2, preferred_element_type=jnp.float32)


def _transform(node_states, w, ka):
    n = node_states.shape[0]
    blk = 1000
    grid = n // blk
    return pl.pallas_call(
        _mm_body,
        grid=(grid,),
        in_specs=[
            pl.BlockSpec((blk, D), lambda i: (i, 0)),
            pl.BlockSpec((D, D), lambda i: (0, 0)),
            pl.BlockSpec((2 * D, 1), lambda i: (0, 0)),
        ],
        out_specs=[
            pl.BlockSpec((blk, D), lambda i: (i, 0)),
            pl.BlockSpec((blk, 2), lambda i: (i, 0)),
        ],
        out_shape=[
            jax.ShapeDtypeStruct((n, D), jnp.bfloat16),
            jax.ShapeDtypeStruct((n, 2), jnp.float32),
        ],
    )(node_states, w, ka)


def _sc_body(tgt_hbm, src_hbm, st_hbm, ss_hbm, h_hbm,
             acc_hbm, sum_hbm,
             tgt_v, src_v, stg_v, ssg_v, e_v, rows_v, rows_w, sbuf, zsum_v,
             acc_sh, sum_sh, sem0, sem1):
    c = lax.axis_index("c")
    s = lax.axis_index("s")
    wid = c * NS + s

    zero16 = jnp.zeros((16,), jnp.float32)

    def zrow(j, carry):
        for k in range(D // 16):
            sbuf[j, pl.ds(k * 16, 16)] = zero16
        return carry

    lax.fori_loop(0, EB, zrow, 0)

    def zs(j, carry):
        zsum_v[pl.ds(j * 16, 16)] = zero16
        return carry

    lax.fori_loop(0, RPT // 16, zs, 0)

    for r in range(RPT // EB):
        pltpu.sync_copy(sbuf, acc_sh.at[pl.ds(s * RPT + r * EB, EB)])
    pltpu.sync_copy(zsum_v, sum_sh.at[pl.ds(s * RPT, RPT)])
    plsc.subcore_barrier()

    def compute_e(g, par):
        def chunk(k, carry2):
            base = k * 16
            raw = stg_v[par, pl.ds(base, 16)] + ssg_v[par, pl.ds(base, 16)]
            lr = jnp.where(raw >= 0.0, raw, raw * 0.2)
            e_v[g, pl.ds(base, 16)] = jnp.exp(jnp.clip(lr, -2.0, 2.0))
            return carry2

        lax.fori_loop(0, EB // 16, chunk, 0)

    def scale_and_scatter(g, buf):
        def chunk(k, carry2):
            base = k * 16
            e16 = e_v[g, pl.ds(base, 16)]
            for lane in range(16):
                av = jnp.full((16,), e16[lane], jnp.float32)
                avh = av * (1.0 - 2.0 ** -8)
                j = base + lane
                for kk in range(D // 32):
                    w = plsc.bitcast(buf[j, pl.ds(kk * 16, 16)], jnp.uint32)
                    lo = plsc.bitcast(w << 16, jnp.float32)
                    hi = plsc.bitcast(w, jnp.float32)
                    sbuf[j, pl.ds(kk * 32, 16)] = lo * av
                    sbuf[j, pl.ds(kk * 32 + 16, 16)] = hi * avh
            return carry2

        lax.fori_loop(0, EB // 16, chunk, 0)
        pltpu.sync_copy(sbuf, acc_sh.at[tgt_v.at[g]], add=True)
        pltpu.sync_copy(e_v.at[g], sum_sh.at[tgt_v.at[g]], add=True)

    def issue_gathers(g, par, buf, sem_):
        pltpu.async_copy(h_hbm.at[src_v.at[g]], buf, sem_)
        pltpu.async_copy(st_hbm.at[tgt_v.at[g]], stg_v.at[par], sem_)
        pltpu.async_copy(ss_hbm.at[src_v.at[g]], ssg_v.at[par], sem_)

    def wait_gathers(g, par, buf, sem_):
        pltpu.make_async_copy(h_hbm.at[src_v.at[g]], buf, sem_).wait()
        pltpu.make_async_copy(st_hbm.at[tgt_v.at[g]], stg_v.at[par], sem_).wait()
        pltpu.make_async_copy(ss_hbm.at[src_v.at[g]], ssg_v.at[par], sem_).wait()

    def group_body(go, carry):
        base_row = wid * NB + go * G
        pltpu.sync_copy(tgt_hbm.at[pl.ds(base_row, G)], tgt_v)
        pltpu.sync_copy(src_hbm.at[pl.ds(base_row, G)], src_v)

        issue_gathers(0, 0, rows_v, sem0)

        def pair_body(p, carry1):
            b0 = 2 * p
            b1 = b0 + 1
            issue_gathers(b1, 1, rows_w, sem1)
            wait_gathers(b0, 0, rows_v, sem0)
            compute_e(b0, 0)
            scale_and_scatter(b0, rows_v)

            @pl.when(p < G // 2 - 1)
            def _():
                issue_gathers(b1 + 1, 0, rows_v, sem0)

            wait_gathers(b1, 1, rows_w, sem1)
            compute_e(b1, 1)
            scale_and_scatter(b1, rows_w)
            return carry1

        lax.fori_loop(0, G // 2, pair_body, 0)
        return carry

    lax.fori_loop(0, NB // G, group_body, 0)
    plsc.subcore_barrier()

    for r in range(RPT // EB):
        pltpu.sync_copy(acc_sh.at[pl.ds(s * RPT + r * EB, EB)],
                        acc_hbm.at[c, pl.ds(s * RPT + r * EB, EB)])
    pltpu.sync_copy(sum_sh.at[pl.ds(s * RPT, RPT)],
                    sum_hbm.at[c, pl.ds(s * RPT, RPT)])


def _sc_edge_pass(tgt2d, src2d, st_pad, ss_pad, h):
    mesh = plsc.VectorSubcoreMesh(core_axis_name="c", subcore_axis_name="s")
    return pl.kernel(
        _sc_body,
        out_type=[
            jax.ShapeDtypeStruct((NC, N_PAD, D), jnp.float32),
            jax.ShapeDtypeStruct((NC, N_PAD), jnp.float32),
        ],
        mesh=mesh,
        compiler_params=pltpu.CompilerParams(needs_layout_passes=False,
                                             use_tc_tiling_on_sc=False),
        scratch_types=[
            pltpu.VMEM((G, EB), jnp.int32),
            pltpu.VMEM((G, EB), jnp.int32),
            pltpu.VMEM((2, EB), jnp.float32),
            pltpu.VMEM((2, EB), jnp.float32),
            pltpu.VMEM((G, EB), jnp.float32),
            pltpu.VMEM((EB, D // 2), jnp.int32),
            pltpu.VMEM((EB, D // 2), jnp.int32),
            pltpu.VMEM((EB, D), jnp.float32),
            pltpu.VMEM((RPT,), jnp.float32),
            pltpu.VMEM_SHARED((N_PAD, D), jnp.float32),
            pltpu.VMEM_SHARED((N_PAD,), jnp.float32),
            pltpu.SemaphoreType.DMA,
            pltpu.SemaphoreType.DMA,
        ],
    )(tgt2d, src2d, st_pad, ss_pad, h)


def _norm_body(a0_ref, a1_ref, s0_ref, s1_ref, p_ref, o_ref):
    tot = s0_ref[...] + s1_ref[...] + 1e-9
    acc = jnp.dot(a0_ref[...] + a1_ref[...], p_ref[...],
                  preferred_element_type=jnp.float32)
    o_ref[...] = acc / tot


def _normalize(acc, sums, p):
    blk = 1024
    grid = N_PAD // blk
    return pl.pallas_call(
        _norm_body,
        grid=(grid,),
        in_specs=[
            pl.BlockSpec((blk, D), lambda i: (i, 0)),
            pl.BlockSpec((blk, D), lambda i: (i, 0)),
            pl.BlockSpec((blk, 1), lambda i: (i, 0)),
            pl.BlockSpec((blk, 1), lambda i: (i, 0)),
            pl.BlockSpec((D, D), lambda i: (0, 0)),
        ],
        out_specs=pl.BlockSpec((blk, D), lambda i: (i, 0)),
        out_shape=jax.ShapeDtypeStruct((N_PAD, D), jnp.float32),
    )(acc[0], acc[1], sums[0][:, None], sums[1][:, None], p)


_SC_PERM = np.concatenate(
    [np.concatenate([np.arange(32 * kk, 32 * (kk + 1), 2),
                     np.arange(32 * kk + 1, 32 * (kk + 1), 2)])
     for kk in range(D // 32)])
_P_UNPERM = np.zeros((D, D), np.float32)
_P_UNPERM[np.arange(D), _SC_PERM] = 1.0


@jax.jit
def kernel(node_states, edges, kernel, kernel_attention):
    h, scores = _transform(node_states, kernel, kernel_attention)

    h32 = lax.bitcast_convert_type(
        h.reshape(h.shape[0], D // 2, 2), jnp.int32)

    tgt = edges[:, 0].astype(jnp.int32)
    src = edges[:, 1].astype(jnp.int32)
    n_fill = E_PAD - N_EDGES
    fill_t = N_NODES + (jnp.arange(n_fill, dtype=jnp.int32) % (N_PAD - N_NODES))
    tgt2d = jnp.concatenate([tgt, fill_t]).reshape(NW * NB, EB)
    src2d = jnp.concatenate([src, jnp.zeros((n_fill,), jnp.int32)]).reshape(
        NW * NB, EB)

    st_pad = jnp.pad(scores[:, 0], (0, N_PAD - N_NODES))
    ss_pad = jnp.pad(scores[:, 1], (0, N_PAD - N_NODES))

    acc, sums = _sc_edge_pass(tgt2d, src2d, st_pad, ss_pad, h32)
    out = _normalize(acc, sums, jnp.asarray(_P_UNPERM))
    return out[:N_NODES]


# speedup vs baseline: 10.9550x; 1.0022x over previous
import functools

import numpy as np
import jax
import jax.numpy as jnp
from jax import lax
from jax.experimental import pallas as pl
from jax.experimental.pallas import tpu as pltpu
from jax.experimental.pallas import tpu_sc as plsc

N_NODES = 10000
N_EDGES = 320000
D = 128

NC = 2
NS = 16
NW = NC * NS

EB = 128
E_PAD = 327680
NB = E_PAD // (NW * EB)
N_PAD = 10240
RPT = N_PAD // NS
G = 8


def _mm_body(ns_ref, w_ref, ka_ref, h_ref, sc_ref):
    h = jnp.dot(ns_ref[...], w_ref[...], preferred_element_type=jnp.float32)
    h_ref[...] = h.astype(jnp.bfloat16)
    a2 = jnp.concatenate([ka_ref[0:D, :], ka_ref[D:2 * D, :]], axis=1)
    sc_ref[...] = jnp.dot(h, a2, preferred_element_type=jnp.float32)


def _transform(node_states, w, ka):
    n = node_states.shape[0]
    blk = 1000
    grid = n // blk
    return pl.pallas_call(
        _mm_body,
        grid=(grid,),
        in_specs=[
            pl.BlockSpec((blk, D), lambda i: (i, 0)),
            pl.BlockSpec((D, D), lambda i: (0, 0)),
            pl.BlockSpec((2 * D, 1), lambda i: (0, 0)),
        ],
        out_specs=[
            pl.BlockSpec((blk, D), lambda i: (i, 0)),
            pl.BlockSpec((blk, 2), lambda i: (i, 0)),
        ],
        out_shape=[
            jax.ShapeDtypeStruct((n, D), jnp.bfloat16),
            jax.ShapeDtypeStruct((n, 2), jnp.float32),
        ],
    )(node_states, w, ka)


def _sc_body(tgt_hbm, src_hbm, st_hbm, ss_hbm, h_hbm,
             acc_hbm, sum_hbm,
             tgt_v, src_v, stg_v, ssg_v, e_v, rows_v, rows_w, sbuf, zsum_v,
             acc_sh, sum_sh, sem0, sem1, sem2):
    c = lax.axis_index("c")
    s = lax.axis_index("s")
    wid = c * NS + s

    zero16 = jnp.zeros((16,), jnp.float32)

    def zrow(j, carry):
        for k in range(D // 16):
            sbuf[j, pl.ds(k * 16, 16)] = zero16
        return carry

    lax.fori_loop(0, EB, zrow, 0)

    def zs(j, carry):
        zsum_v[pl.ds(j * 16, 16)] = zero16
        return carry

    lax.fori_loop(0, RPT // 16, zs, 0)

    for r in range(RPT // EB):
        pltpu.sync_copy(sbuf, acc_sh.at[pl.ds(s * RPT + r * EB, EB)])
    pltpu.sync_copy(zsum_v, sum_sh.at[pl.ds(s * RPT, RPT)])
    plsc.subcore_barrier()

    def compute_e(g, par):
        def chunk(k, carry2):
            base = k * 16
            raw = stg_v[par, pl.ds(base, 16)] + ssg_v[par, pl.ds(base, 16)]
            lr = jnp.where(raw >= 0.0, raw, raw * 0.2)
            e_v[g, pl.ds(base, 16)] = jnp.exp(jnp.clip(lr, -2.0, 2.0))
            return carry2

        lax.fori_loop(0, EB // 16, chunk, 0)

    def scale_and_scatter(g, buf):
        def scale_half(h):
            def chunk(k, carry2):
                base = h * (EB // 2) + k * 16
                e16 = e_v[g, pl.ds(base, 16)]
                for lane in range(16):
                    av = jnp.full((16,), e16[lane], jnp.float32)
                    avh = av * (1.0 - 2.0 ** -8)
                    j = base + lane
                    for kk in range(D // 32):
                        w = plsc.bitcast(buf[j, pl.ds(kk * 16, 16)],
                                         jnp.uint32)
                        lo = plsc.bitcast(w << 16, jnp.float32)
                        hi = plsc.bitcast(w, jnp.float32)
                        sbuf[j, pl.ds(kk * 32, 16)] = lo * av
                        sbuf[j, pl.ds(kk * 32 + 16, 16)] = hi * avh
                return carry2

            lax.fori_loop(0, EB // 32, chunk, 0)

        d_e = pltpu.async_copy(e_v.at[g], sum_sh.at[tgt_v.at[g]], sem2,
                               add=True)
        scale_half(0)
        d_h0 = pltpu.async_copy(
            sbuf.at[pl.ds(0, EB // 2)],
            acc_sh.at[tgt_v.at[g, pl.ds(0, EB // 2)]], sem2, add=True)
        scale_half(1)
        d_h1 = pltpu.async_copy(
            sbuf.at[pl.ds(EB // 2, EB // 2)],
            acc_sh.at[tgt_v.at[g, pl.ds(EB // 2, EB // 2)]], sem2, add=True)
        d_e.wait()
        d_h0.wait()
        d_h1.wait()

    def issue_gathers(g, par, buf, sem_):
        pltpu.async_copy(h_hbm.at[src_v.at[g]], buf, sem_)
        pltpu.async_copy(st_hbm.at[tgt_v.at[g]], stg_v.at[par], sem_)
        pltpu.async_copy(ss_hbm.at[src_v.at[g]], ssg_v.at[par], sem_)

    def wait_gathers(g, par, buf, sem_):
        pltpu.make_async_copy(h_hbm.at[src_v.at[g]], buf, sem_).wait()
        pltpu.make_async_copy(st_hbm.at[tgt_v.at[g]], stg_v.at[par], sem_).wait()
        pltpu.make_async_copy(ss_hbm.at[src_v.at[g]], ssg_v.at[par], sem_).wait()

    def group_body(go, carry):
        base_row = wid * NB + go * G
        pltpu.sync_copy(tgt_hbm.at[pl.ds(base_row, G)], tgt_v)
        pltpu.sync_copy(src_hbm.at[pl.ds(base_row, G)], src_v)

        issue_gathers(0, 0, rows_v, sem0)

        def pair_body(p, carry1):
            b0 = 2 * p
            b1 = b0 + 1
            issue_gathers(b1, 1, rows_w, sem1)
            wait_gathers(b0, 0, rows_v, sem0)
            compute_e(b0, 0)
            scale_and_scatter(b0, rows_v)

            @pl.when(p < G // 2 - 1)
            def _():
                issue_gathers(b1 + 1, 0, rows_v, sem0)

            wait_gathers(b1, 1, rows_w, sem1)
            compute_e(b1, 1)
            scale_and_scatter(b1, rows_w)
            return carry1

        lax.fori_loop(0, G // 2, pair_body, 0)
        return carry

    lax.fori_loop(0, NB // G, group_body, 0)
    plsc.subcore_barrier()

    for r in range(RPT // EB):
        pltpu.sync_copy(acc_sh.at[pl.ds(s * RPT + r * EB, EB)],
                        acc_hbm.at[c, pl.ds(s * RPT + r * EB, EB)])
    pltpu.sync_copy(sum_sh.at[pl.ds(s * RPT, RPT)],
                    sum_hbm.at[c, pl.ds(s * RPT, RPT)])


def _sc_edge_pass(tgt2d, src2d, st_pad, ss_pad, h):
    mesh = plsc.VectorSubcoreMesh(core_axis_name="c", subcore_axis_name="s")
    return pl.kernel(
        _sc_body,
        out_type=[
            jax.ShapeDtypeStruct((NC, N_PAD, D), jnp.float32),
            jax.ShapeDtypeStruct((NC, N_PAD), jnp.float32),
        ],
        mesh=mesh,
        compiler_params=pltpu.CompilerParams(needs_layout_passes=False,
                                             use_tc_tiling_on_sc=False),
        scratch_types=[
            pltpu.VMEM((G, EB), jnp.int32),
            pltpu.VMEM((G, EB), jnp.int32),
            pltpu.VMEM((2, EB), jnp.float32),
            pltpu.VMEM((2, EB), jnp.float32),
            pltpu.VMEM((G, EB), jnp.float32),
            pltpu.VMEM((EB, D // 2), jnp.int32),
            pltpu.VMEM((EB, D // 2), jnp.int32),
            pltpu.VMEM((EB, D), jnp.float32),
            pltpu.VMEM((RPT,), jnp.float32),
            pltpu.VMEM_SHARED((N_PAD, D), jnp.float32),
            pltpu.VMEM_SHARED((N_PAD,), jnp.float32),
            pltpu.SemaphoreType.DMA,
            pltpu.SemaphoreType.DMA,
            pltpu.SemaphoreType.DMA,
        ],
    )(tgt2d, src2d, st_pad, ss_pad, h)


def _norm_body(a0_ref, a1_ref, s0_ref, s1_ref, p_ref, o_ref):
    tot = s0_ref[...] + s1_ref[...] + 1e-9
    acc = jnp.dot(a0_ref[...] + a1_ref[...], p_ref[...],
                  preferred_element_type=jnp.float32)
    o_ref[...] = acc / tot


def _normalize(acc, sums, p):
    blk = 1024
    grid = N_PAD // blk
    return pl.pallas_call(
        _norm_body,
        grid=(grid,),
        in_specs=[
            pl.BlockSpec((blk, D), lambda i: (i, 0)),
            pl.BlockSpec((blk, D), lambda i: (i, 0)),
            pl.BlockSpec((blk, 1), lambda i: (i, 0)),
            pl.BlockSpec((blk, 1), lambda i: (i, 0)),
            pl.BlockSpec((D, D), lambda i: (0, 0)),
        ],
        out_specs=pl.BlockSpec((blk, D), lambda i: (i, 0)),
        out_shape=jax.ShapeDtypeStruct((N_PAD, D), jnp.float32),
    )(acc[0], acc[1], sums[0][:, None], sums[1][:, None], p)


_SC_PERM = np.concatenate(
    [np.concatenate([np.arange(32 * kk, 32 * (kk + 1), 2),
                     np.arange(32 * kk + 1, 32 * (kk + 1), 2)])
     for kk in range(D // 32)])
_P_UNPERM = np.zeros((D, D), np.float32)
_P_UNPERM[np.arange(D), _SC_PERM] = 1.0


@jax.jit
def kernel(node_states, edges, kernel, kernel_attention):
    h, scores = _transform(node_states, kernel, kernel_attention)

    h32 = lax.bitcast_convert_type(
        h.reshape(h.shape[0], D // 2, 2), jnp.int32)

    tgt = edges[:, 0].astype(jnp.int32)
    src = edges[:, 1].astype(jnp.int32)
    n_fill = E_PAD - N_EDGES
    fill_t = N_NODES + (jnp.arange(n_fill, dtype=jnp.int32) % (N_PAD - N_NODES))
    tgt2d = jnp.concatenate([tgt, fill_t]).reshape(NW * NB, EB)
    src2d = jnp.concatenate([src, jnp.zeros((n_fill,), jnp.int32)]).reshape(
        NW * NB, EB)

    st_pad = jnp.pad(scores[:, 0], (0, N_PAD - N_NODES))
    ss_pad = jnp.pad(scores[:, 1], (0, N_PAD - N_NODES))

    acc, sums = _sc_edge_pass(tgt2d, src2d, st_pad, ss_pad, h32)
    out = _normalize(acc, sums, jnp.asarray(_P_UNPERM))
    return out[:N_NODES]


# speedup vs baseline: 11.2043x; 1.0228x over previous
import functools

import numpy as np
import jax
import jax.numpy as jnp
from jax import lax
from jax.experimental import pallas as pl
from jax.experimental.pallas import tpu as pltpu
from jax.experimental.pallas import tpu_sc as plsc

N_NODES = 10000
N_EDGES = 320000
D = 128

NC = 2
NS = 16
NW = NC * NS

EB = 128
E_PAD = 327680
NB = E_PAD // (NW * EB)
N_PAD = 10240
RPT = N_PAD // NS
G = 8


def _mm_body(ns_ref, w_ref, ka_ref, h_ref, sc_ref):
    h = jnp.dot(ns_ref[...], w_ref[...], preferred_element_type=jnp.float32)
    h_ref[...] = h.astype(jnp.bfloat16)
    a2 = jnp.concatenate([ka_ref[0:D, :], ka_ref[D:2 * D, :]], axis=1)
    sc_ref[...] = jnp.dot(h, a2, preferred_element_type=jnp.float32)


def _transform(node_states, w, ka):
    n = node_states.shape[0]
    blk = 1000
    grid = n // blk
    return pl.pallas_call(
        _mm_body,
        grid=(grid,),
        in_specs=[
            pl.BlockSpec((blk, D), lambda i: (i, 0)),
            pl.BlockSpec((D, D), lambda i: (0, 0)),
            pl.BlockSpec((2 * D, 1), lambda i: (0, 0)),
        ],
        out_specs=[
            pl.BlockSpec((blk, D), lambda i: (i, 0)),
            pl.BlockSpec((blk, 2), lambda i: (i, 0)),
        ],
        out_shape=[
            jax.ShapeDtypeStruct((n, D), jnp.bfloat16),
            jax.ShapeDtypeStruct((n, 2), jnp.float32),
        ],
    )(node_states, w, ka)


def _sc_body(tgt_hbm, src_hbm, tab_hbm, h_hbm,
             acc_hbm, sum_hbm,
             tgt_v, src_v, e_v, rows_v, rows_w, sbuf, zsum_v, tab_v,
             acc_sh, sum_sh, sem0, sem1, sem2):
    c = lax.axis_index("c")
    s = lax.axis_index("s")
    wid = c * NS + s

    zero16 = jnp.zeros((16,), jnp.float32)

    def zrow(j, carry):
        for k in range(D // 16):
            sbuf[j, pl.ds(k * 16, 16)] = zero16
        return carry

    lax.fori_loop(0, EB, zrow, 0)

    def zs(j, carry):
        zsum_v[pl.ds(j * 16, 16)] = zero16
        return carry

    lax.fori_loop(0, RPT // 16, zs, 0)

    for r in range(RPT // EB):
        pltpu.sync_copy(sbuf, acc_sh.at[pl.ds(s * RPT + r * EB, EB)])
    pltpu.sync_copy(zsum_v, sum_sh.at[pl.ds(s * RPT, RPT)])
    pltpu.sync_copy(tab_hbm, tab_v)
    plsc.subcore_barrier()

    def compute_e(g):
        def chunk(k, carry2):
            base = k * 16
            ti = tgt_v[g, pl.ds(base, 16)]
            si = src_v[g, pl.ds(base, 16)]
            wt = plsc.load_gather(tab_v, [ti])
            ws = plsc.load_gather(tab_v, [si])
            st16 = plsc.bitcast(wt, jnp.float32) * (1.0 - 2.0 ** -8)
            ss16 = plsc.bitcast(ws << 16, jnp.float32)
            raw = st16 + ss16
            lr = jnp.where(raw >= 0.0, raw, raw * 0.2)
            e_v[g, pl.ds(base, 16)] = jnp.exp(jnp.clip(lr, -2.0, 2.0))
            return carry2

        lax.fori_loop(0, EB // 16, chunk, 0)

    def scale_and_scatter(g, buf):
        def scale_half(h):
            def chunk(k, carry2):
                base = h * (EB // 2) + k * 16
                e16 = e_v[g, pl.ds(base, 16)]
                for lane in range(16):
                    av = jnp.full((16,), e16[lane], jnp.float32)
                    avh = av * (1.0 - 2.0 ** -8)
                    j = base + lane
                    for kk in range(D // 32):
                        w = plsc.bitcast(buf[j, pl.ds(kk * 16, 16)],
                                         jnp.uint32)
                        lo = plsc.bitcast(w << 16, jnp.float32)
                        hi = plsc.bitcast(w, jnp.float32)
                        sbuf[j, pl.ds(kk * 32, 16)] = lo * av
                        sbuf[j, pl.ds(kk * 32 + 16, 16)] = hi * avh
                return carry2

            lax.fori_loop(0, EB // 32, chunk, 0)

        d_e = pltpu.async_copy(e_v.at[g], sum_sh.at[tgt_v.at[g]], sem2,
                               add=True)
        scale_half(0)
        d_h0 = pltpu.async_copy(
            sbuf.at[pl.ds(0, EB // 2)],
            acc_sh.at[tgt_v.at[g, pl.ds(0, EB // 2)]], sem2, add=True)
        scale_half(1)
        d_h1 = pltpu.async_copy(
            sbuf.at[pl.ds(EB // 2, EB // 2)],
            acc_sh.at[tgt_v.at[g, pl.ds(EB // 2, EB // 2)]], sem2, add=True)
        d_e.wait()
        d_h0.wait()
        d_h1.wait()

    def issue_gathers(g, par, buf, sem_):
        pltpu.async_copy(h_hbm.at[src_v.at[g]], buf, sem_)

    def wait_gathers(g, par, buf, sem_):
        pltpu.make_async_copy(h_hbm.at[src_v.at[g]], buf, sem_).wait()

    def group_body(go, carry):
        base_row = wid * NB + go * G
        pltpu.sync_copy(tgt_hbm.at[pl.ds(base_row, G)], tgt_v)
        pltpu.sync_copy(src_hbm.at[pl.ds(base_row, G)], src_v)

        issue_gathers(0, 0, rows_v, sem0)

        def pair_body(p, carry1):
            b0 = 2 * p
            b1 = b0 + 1
            issue_gathers(b1, 1, rows_w, sem1)
            wait_gathers(b0, 0, rows_v, sem0)
            compute_e(b0)
            scale_and_scatter(b0, rows_v)

            @pl.when(p < G // 2 - 1)
            def _():
                issue_gathers(b1 + 1, 0, rows_v, sem0)

            wait_gathers(b1, 1, rows_w, sem1)
            compute_e(b1)
            scale_and_scatter(b1, rows_w)
            return carry1

        lax.fori_loop(0, G // 2, pair_body, 0)
        return carry

    lax.fori_loop(0, NB // G, group_body, 0)
    plsc.subcore_barrier()

    for r in range(RPT // EB):
        pltpu.sync_copy(acc_sh.at[pl.ds(s * RPT + r * EB, EB)],
                        acc_hbm.at[c, pl.ds(s * RPT + r * EB, EB)])
    pltpu.sync_copy(sum_sh.at[pl.ds(s * RPT, RPT)],
                    sum_hbm.at[c, pl.ds(s * RPT, RPT)])


def _sc_edge_pass(tgt2d, src2d, tab, h):
    mesh = plsc.VectorSubcoreMesh(core_axis_name="c", subcore_axis_name="s")
    return pl.kernel(
        _sc_body,
        out_type=[
            jax.ShapeDtypeStruct((NC, N_PAD, D), jnp.float32),
            jax.ShapeDtypeStruct((NC, N_PAD), jnp.float32),
        ],
        mesh=mesh,
        compiler_params=pltpu.CompilerParams(needs_layout_passes=False,
                                             use_tc_tiling_on_sc=False),
        scratch_types=[
            pltpu.VMEM((G, EB), jnp.int32),
            pltpu.VMEM((G, EB), jnp.int32),
            pltpu.VMEM((G, EB), jnp.float32),
            pltpu.VMEM((EB, D // 2), jnp.int32),
            pltpu.VMEM((EB, D // 2), jnp.int32),
            pltpu.VMEM((EB, D), jnp.float32),
            pltpu.VMEM((RPT,), jnp.float32),
            pltpu.VMEM((N_PAD,), jnp.int32),
            pltpu.VMEM_SHARED((N_PAD, D), jnp.float32),
            pltpu.VMEM_SHARED((N_PAD,), jnp.float32),
            pltpu.SemaphoreType.DMA,
            pltpu.SemaphoreType.DMA,
            pltpu.SemaphoreType.DMA,
        ],
    )(tgt2d, src2d, tab, h)


def _norm_body(a0_ref, a1_ref, s0_ref, s1_ref, p_ref, o_ref):
    tot = s0_ref[...] + s1_ref[...] + 1e-9
    acc = jnp.dot(a0_ref[...] + a1_ref[...], p_ref[...],
                  preferred_element_type=jnp.float32)
    o_ref[...] = acc / tot


def _normalize(acc, sums, p):
    blk = 1024
    grid = N_PAD // blk
    return pl.pallas_call(
        _norm_body,
        grid=(grid,),
        in_specs=[
            pl.BlockSpec((blk, D), lambda i: (i, 0)),
            pl.BlockSpec((blk, D), lambda i: (i, 0)),
            pl.BlockSpec((blk, 1), lambda i: (i, 0)),
            pl.BlockSpec((blk, 1), lambda i: (i, 0)),
            pl.BlockSpec((D, D), lambda i: (0, 0)),
        ],
        out_specs=pl.BlockSpec((blk, D), lambda i: (i, 0)),
        out_shape=jax.ShapeDtypeStruct((N_PAD, D), jnp.float32),
    )(acc[0], acc[1], sums[0][:, None], sums[1][:, None], p)


_SC_PERM = np.concatenate(
    [np.concatenate([np.arange(32 * kk, 32 * (kk + 1), 2),
                     np.arange(32 * kk + 1, 32 * (kk + 1), 2)])
     for kk in range(D // 32)])
_P_UNPERM = np.zeros((D, D), np.float32)
_P_UNPERM[np.arange(D), _SC_PERM] = 1.0


@jax.jit
def kernel(node_states, edges, kernel, kernel_attention):
    h, scores = _transform(node_states, kernel, kernel_attention)

    h32 = lax.bitcast_convert_type(
        h.reshape(h.shape[0], D // 2, 2), jnp.int32)

    tgt = edges[:, 0].astype(jnp.int32)
    src = edges[:, 1].astype(jnp.int32)
    n_fill = E_PAD - N_EDGES
    fill_t = N_NODES + (jnp.arange(n_fill, dtype=jnp.int32) % (N_PAD - N_NODES))
    tgt2d = jnp.concatenate([tgt, fill_t]).reshape(NW * NB, EB)
    src2d = jnp.concatenate([src, jnp.zeros((n_fill,), jnp.int32)]).reshape(
        NW * NB, EB)

    sc16 = lax.bitcast_convert_type(
        scores.astype(jnp.bfloat16), jnp.uint16).astype(jnp.uint32)
    tab = lax.bitcast_convert_type(
        (sc16[:, 0] << 16) | sc16[:, 1], jnp.int32)
    tab = jnp.pad(tab, (0, N_PAD - N_NODES))

    acc, sums = _sc_edge_pass(tgt2d, src2d, tab, h32)
    out = _normalize(acc, sums, jnp.asarray(_P_UNPERM))
    return out[:N_NODES]
